# baseline (speedup 1.0000x reference)
import functools

import numpy as np
import jax
import jax.numpy as jnp
from jax import lax
from jax.experimental import pallas as pl
from jax.experimental.pallas import tpu as pltpu
from jax.experimental.pallas import tpu_sc as plsc

_N_ATOMS = 10000
_E = 160000
_N_SPECIES = 119
_NR = 8
_NB = 16
_RMAX = 6.0

_NW = 32
_EP = 163840
_EPW = _EP // _NW
_NCHUNK = _EPW // 128
_CH2 = 64
_NCHUNK2 = _EPW // _CH2
_NMOM = 20
_NRC = _NR * _NMOM
_AP = 10240
_ROWS_PER_TILE = _AP // 16

_BETTA = float(_NB * _NB) / (_RMAX * _RMAX)
_RAD_NORM = float((2.0 * _BETTA / np.pi) ** 0.75)

_EB = 4096
_GE = _EP // _EB
_AB_SUB = 8
_GA = _AP // (_AB_SUB * 128)

_TRIL2 = [(i, j) for i in range(_NR) for j in range(i + 1)]
_TRIL3 = [(i, j, k) for i in range(_NR) for j in range(i + 1) for k in range(j + 1)]
_S2 = {(0, 0): 0, (0, 1): 1, (0, 2): 2, (1, 1): 3, (1, 2): 4, (2, 2): 5}
_S3 = {(0, 0, 0): 0, (0, 0, 1): 1, (0, 0, 2): 2, (0, 1, 1): 3, (0, 1, 2): 4,
       (0, 2, 2): 5, (1, 1, 1): 6, (1, 1, 2): 7, (1, 2, 2): 8, (2, 2, 2): 9}
_W2 = [1.0, 2.0, 2.0, 1.0, 2.0, 1.0]
_W3 = [1.0, 3.0, 3.0, 3.0, 6.0, 3.0, 1.0, 3.0, 3.0, 1.0]

_NOUT = 8 + 3 * len(_TRIL2) + len(_TRIL3) + 2 * 8 * len(_TRIL2) + 512


def _sc_pairs(z, idx_i, idx_j):
  mesh = plsc.VectorSubcoreMesh(core_axis_name="c", subcore_axis_name="s")

  @functools.partial(
      pl.kernel,
      mesh=mesh,
      compiler_params=pltpu.CompilerParams(needs_layout_passes=False),
      out_type=jax.ShapeDtypeStruct((_EP,), jnp.int32),
      scratch_types=[
          pltpu.VMEM((_N_ATOMS,), jnp.int32),
          pltpu.VMEM((_EPW,), jnp.int32),
          pltpu.VMEM((_EPW,), jnp.int32),
          pltpu.VMEM((_EPW,), jnp.int32),
      ],
  )
  def k(z_hbm, ii_hbm, ij_hbm, out_hbm, z_v, ii_v, ij_v, pair_v):
    cid = lax.axis_index("c")
    sid = lax.axis_index("s")
    wid = cid * 16 + sid
    base = wid * _EPW
    pltpu.sync_copy(z_hbm, z_v)
    pltpu.sync_copy(ii_hbm.at[pl.ds(base, _EPW)], ii_v)
    pltpu.sync_copy(ij_hbm.at[pl.ds(base, _EPW)], ij_v)

    def grp(g, carry):
      vi = ii_v[pl.ds(g * 16, 16)]
      vj = ij_v[pl.ds(g * 16, 16)]
      zi = plsc.load_gather(z_v, [vi])
      zj = plsc.load_gather(z_v, [vj])
      pair_v[pl.ds(g * 16, 16)] = zi * _N_SPECIES + zj
      return carry

    lax.fori_loop(0, _EPW // 16, grp, 0)
    pltpu.sync_copy(pair_v, out_hbm.at[pl.ds(base, _EPW)])

  return k(z, idx_i, idx_j)


_WROWS = 14208
_GCH = 40
_NGCH = _EPW // _GCH


def _sc_gather(w_rows, pairs):
  mesh = plsc.VectorSubcoreMesh(core_axis_name="c", subcore_axis_name="s")

  @functools.partial(
      pl.kernel,
      mesh=mesh,
      compiler_params=pltpu.CompilerParams(use_tc_tiling_on_sc=False),
      out_type=jax.ShapeDtypeStruct((_EP, 128), jnp.float32),
      scratch_types=[
          pltpu.VMEM((_NGCH, _GCH), jnp.int32),
          pltpu.VMEM((2, _GCH, 128), jnp.float32),
          pltpu.VMEM_SHARED((_WROWS, 128), jnp.float32),
          pltpu.SemaphoreType.DMA,
          pltpu.SemaphoreType.DMA,
          pltpu.SemaphoreType.DMA,
          pltpu.SemaphoreType.DMA,
      ],
  )
  def k(w_hbm, pair_hbm, out_hbm, idxall, rows2, w_sh, gs0, gs1, ws0, ws1):
    cid = lax.axis_index("c")
    sid = lax.axis_index("s")
    wid = cid * 16 + sid
    base = wid * _EPW
    stripe = _WROWS // 16
    pltpu.sync_copy(w_hbm.at[pl.ds(sid * stripe, stripe)],
                    w_sh.at[pl.ds(sid * stripe, stripe)])
    pltpu.sync_copy(pair_hbm.at[pl.ds(wid * _NGCH, _NGCH)], idxall)
    plsc.subcore_barrier()
    gsem = [gs0, gs1]
    wsem = [ws0, ws1]
    gh = [None] * _NGCH
    wh = [None] * _NGCH
    for kc in range(_NGCH + 1):
      if kc < _NGCH:
        b = kc % 2
        if kc >= 2:
          wh[kc - 2].wait()
        gh[kc] = pltpu.async_copy(w_sh.at[idxall.at[kc]], rows2.at[b],
                                  gsem[b])
      if kc >= 1:
        p = kc - 1
        pb = p % 2
        gh[p].wait()
        wh[p] = pltpu.async_copy(rows2.at[pb],
                                 out_hbm.at[pl.ds(base + p * _GCH, _GCH)],
                                 wsem[pb])
    wh[_NGCH - 2].wait()
    wh[_NGCH - 1].wait()

  return k(w_rows, pairs)


def _sc_scatter(medge, idx_j2, zeros_init):
  mesh = plsc.VectorSubcoreMesh(core_axis_name="c", subcore_axis_name="s")

  @functools.partial(
      pl.kernel,
      mesh=mesh,
      compiler_params=pltpu.CompilerParams(use_tc_tiling_on_sc=False),
      out_type=jax.ShapeDtypeStruct((2, _AP, _NRC), jnp.float32),
      scratch_types=[
          pltpu.VMEM((_NCHUNK2, _CH2), jnp.int32),
          pltpu.VMEM((2, _CH2, _NRC), jnp.float32),
          pltpu.VMEM_SHARED((_AP, _NRC), jnp.float32),
          pltpu.SemaphoreType.DMA,
          pltpu.SemaphoreType.DMA,
          pltpu.SemaphoreType.DMA,
          pltpu.SemaphoreType.DMA,
          pltpu.SemaphoreType.DMA,
      ],
  )
  def k(me_hbm, ij_hbm, z0_hbm, out_hbm, idxall, rows2, acc_sh,
        is0, rs0, rs1, ss0, ss1):
    cid = lax.axis_index("c")
    sid = lax.axis_index("s")
    wid = cid * 16 + sid
    r0 = sid * _ROWS_PER_TILE
    ebase = wid * _EPW
    pltpu.async_copy(ij_hbm.at[pl.ds(wid * _NCHUNK2, _NCHUNK2)], idxall,
                     is0).wait()
    pltpu.sync_copy(z0_hbm.at[pl.ds(r0, _ROWS_PER_TILE)],
                    acc_sh.at[pl.ds(r0, _ROWS_PER_TILE)])
    plsc.subcore_barrier()
    rsem = [rs0, rs1]
    ssem = [ss0, ss1]
    rh = [None] * _NCHUNK2
    sh = [None] * _NCHUNK2
    for kc in range(_NCHUNK2 + 1):
      if kc < _NCHUNK2:
        b = kc % 2
        if kc >= 2:
          sh[kc - 2].wait()
        rh[kc] = pltpu.async_copy(me_hbm.at[pl.ds(ebase + kc * _CH2, _CH2)],
                                  rows2.at[b], rsem[b])
      if kc >= 1:
        p = kc - 1
        pb = p % 2
        rh[p].wait()
        sh[p] = pltpu.async_copy(rows2.at[pb], acc_sh.at[idxall.at[p]],
                                 ssem[pb], add=True)
    sh[_NCHUNK2 - 2].wait()
    sh[_NCHUNK2 - 1].wait()
    plsc.subcore_barrier()
    pltpu.sync_copy(acc_sh.at[pl.ds(r0, _ROWS_PER_TILE)],
                    out_hbm.at[cid, pl.ds(r0, _ROWS_PER_TILE)])

  return k(medge, idx_j2, zeros_init)


def _edge_body(x_ref, y_ref, z_ref, ii_ref, ij_ref, c_ref, o_ref):
  x = x_ref[0]
  y = y_ref[0]
  z = z_ref[0]
  dr2 = x * x + y * y + z * z
  dr = jnp.sqrt(dr2)
  inv = 1.0 / (dr + 1e-5)
  dnx = x * inv
  dny = y * inv
  dnz = z * inv
  cut = 0.5 * (jnp.cos(jnp.float32(np.pi / _RMAX) * dr) + 1.0)
  valid = ((dr < _RMAX) & (ii_ref[0] != ij_ref[0])).astype(jnp.float32)
  cut = cut * valid

  c_t = jnp.transpose(c_ref[...])
  drb = jnp.broadcast_to(dr, (_NB, _EB))
  sub = lax.broadcasted_iota(jnp.int32, (_NB, _EB), 0).astype(jnp.float32)
  shifts = (_RMAX / (_NB + 1.0)) * (sub + 1.0)
  t = drb - shifts
  bas = jnp.exp((-_BETTA) * t * t) * _RAD_NORM

  rads = []
  for r in range(_NR):
    pr = c_t[r * _NB:(r + 1) * _NB, :] * bas
    rads.append(jnp.sum(pr, axis=0, keepdims=True) * cut)
  rad8 = jnp.concatenate(rads, axis=0)

  xx = dnx * dnx
  xy = dnx * dny
  xz = dnx * dnz
  yy = dny * dny
  yz = dny * dnz
  zz = dnz * dnz
  gs = [None, dnx, dny, dnz, xx, xy, xz, yy, yz, zz,
        xx * dnx, xx * dny, xx * dnz, dnx * yy, xy * dnz, dnx * zz,
        yy * dny, yy * dnz, dny * zz, zz * dnz]
  blocks = [rad8]
  for c in range(1, _NMOM):
    blocks.append(rad8 * jnp.broadcast_to(gs[c], (_NR, _EB)))
  val = jnp.concatenate(blocks, axis=0)
  o_ref[...] = jnp.transpose(val)


def _tc_edge(xs, ys, zs, ii, ij, coeffs):
  spec1 = pl.BlockSpec((1, 1, _EB), lambda i: (i, 0, 0))
  return pl.pallas_call(
      _edge_body,
      grid=(_GE,),
      in_specs=[spec1, spec1, spec1, spec1, spec1,
                pl.BlockSpec((_EB, 128), lambda i: (i, 0))],
      out_specs=pl.BlockSpec((_EB, _NRC), lambda i: (i, 0)),
      out_shape=jax.ShapeDtypeStruct((_EP, _NRC), jnp.float32),
  )(xs, ys, zs, ii, ij, coeffs)


def _contract_body(m_ref, o_ref):
  def M(r, c):
    row = c * _NR + r
    return m_ref[0, row] + m_ref[1, row]

  m0 = [M(r, 0) for r in range(_NR)]
  m1v = [jnp.stack([M(r, 1 + i) for i in range(3)]) for r in range(_NR)]
  m2c = [[M(r, 4 + c) for c in range(6)] for r in range(_NR)]
  m3c = [[M(r, 10 + c) for c in range(10)] for r in range(_NR)]

  m2v = [jnp.stack(m2c[r]) for r in range(_NR)]
  m3v = [jnp.stack(m3c[r]) for r in range(_NR)]
  m2w = [jnp.stack([m2c[r][c] * _W2[c] for c in range(6)]) for r in range(_NR)]
  m3w = [jnp.stack([m3c[r][c] * _W3[c] for c in range(10)]) for r in range(_NR)]

  m2f = [jnp.stack([jnp.stack([m2c[r][_S2[(min(i, j), max(i, j))]]
                               for j in range(3)]) for i in range(3)])
         for r in range(_NR)]
  m3f = [jnp.stack([jnp.stack([jnp.stack([m3c[r][_S3[tuple(sorted((i, j, k)))]]
                                          for k in range(3)]) for j in range(3)])
                    for i in range(3)]) for r in range(_NR)]

  for r in range(_NR):
    o_ref[r] = m0[r]
  off = _NR
  for p, (r, s) in enumerate(_TRIL2):
    o_ref[off + p] = (m1v[r] * m1v[s]).sum(axis=0)
  off += len(_TRIL2)
  for p, (r, s) in enumerate(_TRIL2):
    o_ref[off + p] = (m2w[r] * m2v[s]).sum(axis=0)
  off += len(_TRIL2)
  for p, (r, s) in enumerate(_TRIL2):
    o_ref[off + p] = (m3w[r] * m3v[s]).sum(axis=0)
  off += len(_TRIL2)

  t2cache = {}
  for q, (r, s, t) in enumerate(_TRIL3):
    if (s, t) not in t2cache:
      t2cache[(s, t)] = (m2f[s][:, None] * m2f[t][None, :]).sum(axis=2)
    o_ref[off + q] = (m2f[r] * t2cache[(s, t)]).sum(axis=(0, 1))
  off += len(_TRIL3)

  for p, (r, s) in enumerate(_TRIL2):
    op1 = m1v[r][:, None] * m1v[s][None, :]
    for t in range(_NR):
      o_ref[off + p * _NR + t] = (op1 * m2f[t]).sum(axis=(0, 1))
  off += _NR * len(_TRIL2)

  for p, (r, s) in enumerate(_TRIL2):
    w6 = (m3f[r][:, :, :, None] * m3f[s][:, :, None, :]).sum(axis=(0, 1))
    for t in range(_NR):
      o_ref[off + p * _NR + t] = (w6 * m2f[t]).sum(axis=(0, 1))
  off += _NR * len(_TRIL2)

  for r in range(_NR):
    for s in range(_NR):
      x7 = (m3f[r] * m2f[s][:, :, None]).sum(axis=(0, 1))
      for t in range(_NR):
        o_ref[off + r * 64 + s * _NR + t] = (x7 * m1v[t]).sum(axis=0)


def _tc_contract(acc_t):
  return pl.pallas_call(
      _contract_body,
      grid=(_GA,),
      in_specs=[pl.BlockSpec((2, _NRC, _AB_SUB, 128), lambda i: (0, 0, i, 0))],
      out_specs=pl.BlockSpec((_NOUT, _AB_SUB, 128), lambda i: (0, i, 0)),
      out_shape=jax.ShapeDtypeStruct((_NOUT, _GA * _AB_SUB, 128), jnp.float32),
  )(acc_t)


def kernel(dr_vec, Z, neighbor_idxs, W):
  f32 = jnp.float32
  w_rows = W.astype(f32).reshape(_N_SPECIES * _N_SPECIES, _NR * _NB)
  w_rows = jnp.pad(w_rows, ((0, _WROWS - _N_SPECIES * _N_SPECIES), (0, 0)))
  z32 = Z.astype(jnp.int32)
  pad = _EP - _E
  iip = jnp.pad(neighbor_idxs[0].astype(jnp.int32), (0, pad))
  ijp = jnp.pad(neighbor_idxs[1].astype(jnp.int32), (0, pad))
  drp = jnp.pad(dr_vec.astype(f32), ((0, pad), (0, 0)))

  pairs = _sc_pairs(z32, iip, ijp)
  coeffs = _sc_gather(w_rows, pairs.reshape(_EP // _GCH, _GCH))

  xs = drp[:, 0].reshape(_GE, 1, _EB)
  ys = drp[:, 1].reshape(_GE, 1, _EB)
  zs = drp[:, 2].reshape(_GE, 1, _EB)
  ii3 = iip.reshape(_GE, 1, _EB)
  ij3 = ijp.reshape(_GE, 1, _EB)
  medge = _tc_edge(xs, ys, zs, ii3, ij3, coeffs)

  ij2 = ijp.reshape(_EP // _CH2, _CH2)
  z0 = jnp.zeros((_AP, _NRC), f32)
  acc = _sc_scatter(medge, ij2, z0)

  acc_t = jnp.transpose(acc, (0, 2, 1)).reshape(2, _NRC, _GA * _AB_SUB, 128)
  out = _tc_contract(acc_t)
  return out.reshape(_NOUT, _AP).T[:_N_ATOMS]


# speedup vs baseline: 1.0112x; 1.0112x over previous
import functools

import numpy as np
import jax
import jax.numpy as jnp
from jax import lax
from jax.experimental import pallas as pl
from jax.experimental.pallas import tpu as pltpu
from jax.experimental.pallas import tpu_sc as plsc

_N_ATOMS = 10000
_E = 160000
_N_SPECIES = 119
_NR = 8
_NB = 16
_RMAX = 6.0

_NW = 32
_EP = 163840
_EPW = _EP // _NW
_NCHUNK = _EPW // 128
_CH2 = 64
_NCHUNK2 = _EPW // _CH2
_NMOM = 20
_NRC = _NR * _NMOM
_AP = 10240
_ROWS_PER_TILE = _AP // 16

_BETTA = float(_NB * _NB) / (_RMAX * _RMAX)
_RAD_NORM = float((2.0 * _BETTA / np.pi) ** 0.75)

_EB = 4096
_GE = _EP // _EB
_AB_SUB = 8
_GA = _AP // (_AB_SUB * 128)

_TRIL2 = [(i, j) for i in range(_NR) for j in range(i + 1)]
_TRIL3 = [(i, j, k) for i in range(_NR) for j in range(i + 1) for k in range(j + 1)]
_S2 = {(0, 0): 0, (0, 1): 1, (0, 2): 2, (1, 1): 3, (1, 2): 4, (2, 2): 5}
_S3 = {(0, 0, 0): 0, (0, 0, 1): 1, (0, 0, 2): 2, (0, 1, 1): 3, (0, 1, 2): 4,
       (0, 2, 2): 5, (1, 1, 1): 6, (1, 1, 2): 7, (1, 2, 2): 8, (2, 2, 2): 9}
_W2 = [1.0, 2.0, 2.0, 1.0, 2.0, 1.0]
_W3 = [1.0, 3.0, 3.0, 3.0, 6.0, 3.0, 1.0, 3.0, 3.0, 1.0]

_NOUT = 8 + 3 * len(_TRIL2) + len(_TRIL3) + 2 * 8 * len(_TRIL2) + 512


def _sc_pairs(z, idx_i, idx_j):
  mesh = plsc.VectorSubcoreMesh(core_axis_name="c", subcore_axis_name="s")

  @functools.partial(
      pl.kernel,
      mesh=mesh,
      compiler_params=pltpu.CompilerParams(needs_layout_passes=False),
      out_type=jax.ShapeDtypeStruct((_EP,), jnp.int32),
      scratch_types=[
          pltpu.VMEM((_N_ATOMS,), jnp.int32),
          pltpu.VMEM((_EPW,), jnp.int32),
          pltpu.VMEM((_EPW,), jnp.int32),
          pltpu.VMEM((_EPW,), jnp.int32),
      ],
  )
  def k(z_hbm, ii_hbm, ij_hbm, out_hbm, z_v, ii_v, ij_v, pair_v):
    cid = lax.axis_index("c")
    sid = lax.axis_index("s")
    wid = cid * 16 + sid
    base = wid * _EPW
    pltpu.sync_copy(z_hbm, z_v)
    pltpu.sync_copy(ii_hbm.at[pl.ds(base, _EPW)], ii_v)
    pltpu.sync_copy(ij_hbm.at[pl.ds(base, _EPW)], ij_v)

    def grp(g, carry):
      vi = ii_v[pl.ds(g * 16, 16)]
      vj = ij_v[pl.ds(g * 16, 16)]
      zi = plsc.load_gather(z_v, [vi])
      zj = plsc.load_gather(z_v, [vj])
      pair_v[pl.ds(g * 16, 16)] = zi * _N_SPECIES + zj
      return carry

    lax.fori_loop(0, _EPW // 16, grp, 0)
    pltpu.sync_copy(pair_v, out_hbm.at[pl.ds(base, _EPW)])

  return k(z, idx_i, idx_j)


_WROWS = 14208
_GCH = 40
_NGCH = _EPW // _GCH


def _sc_gather(w_rows, pairs):
  mesh = plsc.VectorSubcoreMesh(core_axis_name="c", subcore_axis_name="s")

  @functools.partial(
      pl.kernel,
      mesh=mesh,
      compiler_params=pltpu.CompilerParams(use_tc_tiling_on_sc=False),
      out_type=jax.ShapeDtypeStruct((_EP, 128), jnp.float32),
      scratch_types=[
          pltpu.VMEM((_NGCH, _GCH), jnp.int32),
          pltpu.VMEM((2, _GCH, 128), jnp.float32),
          pltpu.VMEM_SHARED((_WROWS, 128), jnp.float32),
          pltpu.SemaphoreType.DMA,
          pltpu.SemaphoreType.DMA,
          pltpu.SemaphoreType.DMA,
          pltpu.SemaphoreType.DMA,
      ],
  )
  def k(w_hbm, pair_hbm, out_hbm, idxall, rows2, w_sh, gs0, gs1, ws0, ws1):
    cid = lax.axis_index("c")
    sid = lax.axis_index("s")
    wid = cid * 16 + sid
    base = wid * _EPW
    stripe = _WROWS // 16
    pltpu.sync_copy(w_hbm.at[pl.ds(sid * stripe, stripe)],
                    w_sh.at[pl.ds(sid * stripe, stripe)])
    pltpu.sync_copy(pair_hbm.at[pl.ds(wid * _NGCH, _NGCH)], idxall)
    plsc.subcore_barrier()
    gsem = [gs0, gs1]
    wsem = [ws0, ws1]
    gh = [None] * _NGCH
    wh = [None] * _NGCH
    for kc in range(_NGCH + 1):
      if kc < _NGCH:
        b = kc % 2
        if kc >= 2:
          wh[kc - 2].wait()
        gh[kc] = pltpu.async_copy(w_sh.at[idxall.at[kc]], rows2.at[b],
                                  gsem[b])
      if kc >= 1:
        p = kc - 1
        pb = p % 2
        gh[p].wait()
        wh[p] = pltpu.async_copy(rows2.at[pb],
                                 out_hbm.at[pl.ds(base + p * _GCH, _GCH)],
                                 wsem[pb])
    wh[_NGCH - 2].wait()
    wh[_NGCH - 1].wait()

  return k(w_rows, pairs)


def _sc_scatter(medge, idx_j2):
  mesh = plsc.VectorSubcoreMesh(core_axis_name="c", subcore_axis_name="s")

  @functools.partial(
      pl.kernel,
      mesh=mesh,
      compiler_params=pltpu.CompilerParams(use_tc_tiling_on_sc=False),
      out_type=jax.ShapeDtypeStruct((2, _AP, _NRC), jnp.float32),
      scratch_types=[
          pltpu.VMEM((_NCHUNK2, _CH2), jnp.int32),
          pltpu.VMEM((2, _CH2, _NRC), jnp.float32),
          pltpu.VMEM_SHARED((_AP, _NRC), jnp.float32),
          pltpu.SemaphoreType.DMA,
          pltpu.SemaphoreType.DMA,
          pltpu.SemaphoreType.DMA,
          pltpu.SemaphoreType.DMA,
          pltpu.SemaphoreType.DMA,
      ],
  )
  def k(me_hbm, ij_hbm, out_hbm, idxall, rows2, acc_sh,
        is0, rs0, rs1, ss0, ss1):
    cid = lax.axis_index("c")
    sid = lax.axis_index("s")
    wid = cid * 16 + sid
    r0 = sid * _ROWS_PER_TILE
    ebase = wid * _EPW
    ijh = pltpu.async_copy(ij_hbm.at[pl.ds(wid * _NCHUNK2, _NCHUNK2)], idxall,
                           is0)
    zvec = jnp.zeros((16,), jnp.float32)

    def zrow(i, carry):
      row = i // (_NRC // 16)
      col = (i % (_NRC // 16)) * 16
      rows2[0, row, pl.ds(col, 16)] = zvec
      return carry

    lax.fori_loop(0, _CH2 * (_NRC // 16), zrow, 0)
    for q in range(_ROWS_PER_TILE // _CH2):
      pltpu.sync_copy(rows2.at[0],
                      acc_sh.at[pl.ds(r0 + q * _CH2, _CH2)])
    ijh.wait()
    plsc.subcore_barrier()
    rsem = [rs0, rs1]
    ssem = [ss0, ss1]
    rh = [None] * _NCHUNK2
    sh = [None] * _NCHUNK2
    for kc in range(_NCHUNK2 + 1):
      if kc < _NCHUNK2:
        b = kc % 2
        if kc >= 2:
          sh[kc - 2].wait()
        rh[kc] = pltpu.async_copy(me_hbm.at[pl.ds(ebase + kc * _CH2, _CH2)],
                                  rows2.at[b], rsem[b])
      if kc >= 1:
        p = kc - 1
        pb = p % 2
        rh[p].wait()
        sh[p] = pltpu.async_copy(rows2.at[pb], acc_sh.at[idxall.at[p]],
                                 ssem[pb], add=True)
    sh[_NCHUNK2 - 2].wait()
    sh[_NCHUNK2 - 1].wait()
    plsc.subcore_barrier()
    pltpu.sync_copy(acc_sh.at[pl.ds(r0, _ROWS_PER_TILE)],
                    out_hbm.at[cid, pl.ds(r0, _ROWS_PER_TILE)])

  return k(medge, idx_j2)


def _edge_body(x_ref, y_ref, z_ref, ii_ref, ij_ref, c_ref, o_ref):
  x = x_ref[0]
  y = y_ref[0]
  z = z_ref[0]
  dr2 = x * x + y * y + z * z
  dr = jnp.sqrt(dr2)
  inv = 1.0 / (dr + 1e-5)
  dnx = x * inv
  dny = y * inv
  dnz = z * inv
  cut = 0.5 * (jnp.cos(jnp.float32(np.pi / _RMAX) * dr) + 1.0)
  valid = ((dr < _RMAX) & (ii_ref[0] != ij_ref[0])).astype(jnp.float32)
  cut = cut * valid

  c_t = jnp.transpose(c_ref[...])
  drb = jnp.broadcast_to(dr, (_NB, _EB))
  sub = lax.broadcasted_iota(jnp.int32, (_NB, _EB), 0).astype(jnp.float32)
  shifts = (_RMAX / (_NB + 1.0)) * (sub + 1.0)
  t = drb - shifts
  bas = jnp.exp((-_BETTA) * t * t) * _RAD_NORM

  rads = []
  for r in range(_NR):
    pr = c_t[r * _NB:(r + 1) * _NB, :] * bas
    rads.append(jnp.sum(pr, axis=0, keepdims=True) * cut)
  rad8 = jnp.concatenate(rads, axis=0)

  xx = dnx * dnx
  xy = dnx * dny
  xz = dnx * dnz
  yy = dny * dny
  yz = dny * dnz
  zz = dnz * dnz
  gs = [None, dnx, dny, dnz, xx, xy, xz, yy, yz, zz,
        xx * dnx, xx * dny, xx * dnz, dnx * yy, xy * dnz, dnx * zz,
        yy * dny, yy * dnz, dny * zz, zz * dnz]
  blocks = [rad8]
  for c in range(1, _NMOM):
    blocks.append(rad8 * jnp.broadcast_to(gs[c], (_NR, _EB)))
  val = jnp.concatenate(blocks, axis=0)
  o_ref[...] = jnp.transpose(val)


def _tc_edge(xs, ys, zs, ii, ij, coeffs):
  spec1 = pl.BlockSpec((1, 1, _EB), lambda i: (i, 0, 0))
  return pl.pallas_call(
      _edge_body,
      grid=(_GE,),
      in_specs=[spec1, spec1, spec1, spec1, spec1,
                pl.BlockSpec((_EB, 128), lambda i: (i, 0))],
      out_specs=pl.BlockSpec((_EB, _NRC), lambda i: (i, 0)),
      out_shape=jax.ShapeDtypeStruct((_EP, _NRC), jnp.float32),
  )(xs, ys, zs, ii, ij, coeffs)


def _contract_body(m_ref, o_ref):
  def M(r, c):
    row = c * _NR + r
    return m_ref[0, row] + m_ref[1, row]

  m0 = [M(r, 0) for r in range(_NR)]
  m1v = [jnp.stack([M(r, 1 + i) for i in range(3)]) for r in range(_NR)]
  m2c = [[M(r, 4 + c) for c in range(6)] for r in range(_NR)]
  m3c = [[M(r, 10 + c) for c in range(10)] for r in range(_NR)]

  m2v = [jnp.stack(m2c[r]) for r in range(_NR)]
  m3v = [jnp.stack(m3c[r]) for r in range(_NR)]
  m2w = [jnp.stack([m2c[r][c] * _W2[c] for c in range(6)]) for r in range(_NR)]
  m3w = [jnp.stack([m3c[r][c] * _W3[c] for c in range(10)]) for r in range(_NR)]

  m2f = [jnp.stack([jnp.stack([m2c[r][_S2[(min(i, j), max(i, j))]]
                               for j in range(3)]) for i in range(3)])
         for r in range(_NR)]
  m3f = [jnp.stack([jnp.stack([jnp.stack([m3c[r][_S3[tuple(sorted((i, j, k)))]]
                                          for k in range(3)]) for j in range(3)])
                    for i in range(3)]) for r in range(_NR)]

  for r in range(_NR):
    o_ref[r] = m0[r]
  off = _NR
  for p, (r, s) in enumerate(_TRIL2):
    o_ref[off + p] = (m1v[r] * m1v[s]).sum(axis=0)
  off += len(_TRIL2)
  for p, (r, s) in enumerate(_TRIL2):
    o_ref[off + p] = (m2w[r] * m2v[s]).sum(axis=0)
  off += len(_TRIL2)
  for p, (r, s) in enumerate(_TRIL2):
    o_ref[off + p] = (m3w[r] * m3v[s]).sum(axis=0)
  off += len(_TRIL2)

  t2cache = {}
  for q, (r, s, t) in enumerate(_TRIL3):
    if (s, t) not in t2cache:
      t2cache[(s, t)] = (m2f[s][:, None] * m2f[t][None, :]).sum(axis=2)
    o_ref[off + q] = (m2f[r] * t2cache[(s, t)]).sum(axis=(0, 1))
  off += len(_TRIL3)

  for p, (r, s) in enumerate(_TRIL2):
    op1 = m1v[r][:, None] * m1v[s][None, :]
    for t in range(_NR):
      o_ref[off + p * _NR + t] = (op1 * m2f[t]).sum(axis=(0, 1))
  off += _NR * len(_TRIL2)

  for p, (r, s) in enumerate(_TRIL2):
    w6 = (m3f[r][:, :, :, None] * m3f[s][:, :, None, :]).sum(axis=(0, 1))
    for t in range(_NR):
      o_ref[off + p * _NR + t] = (w6 * m2f[t]).sum(axis=(0, 1))
  off += _NR * len(_TRIL2)

  for r in range(_NR):
    for s in range(_NR):
      x7 = (m3f[r] * m2f[s][:, :, None]).sum(axis=(0, 1))
      for t in range(_NR):
        o_ref[off + r * 64 + s * _NR + t] = (x7 * m1v[t]).sum(axis=0)


def _tc_contract(acc_t):
  return pl.pallas_call(
      _contract_body,
      grid=(_GA,),
      in_specs=[pl.BlockSpec((2, _NRC, _AB_SUB, 128), lambda i: (0, 0, i, 0))],
      out_specs=pl.BlockSpec((_NOUT, _AB_SUB, 128), lambda i: (0, i, 0)),
      out_shape=jax.ShapeDtypeStruct((_NOUT, _GA * _AB_SUB, 128), jnp.float32),
  )(acc_t)


def kernel(dr_vec, Z, neighbor_idxs, W):
  f32 = jnp.float32
  w_rows = W.astype(f32).reshape(_N_SPECIES * _N_SPECIES, _NR * _NB)
  w_rows = jnp.pad(w_rows, ((0, _WROWS - _N_SPECIES * _N_SPECIES), (0, 0)))
  z32 = Z.astype(jnp.int32)
  pad = _EP - _E
  iip = jnp.pad(neighbor_idxs[0].astype(jnp.int32), (0, pad))
  ijp = jnp.pad(neighbor_idxs[1].astype(jnp.int32), (0, pad))
  drp = jnp.pad(dr_vec.astype(f32), ((0, pad), (0, 0)))

  pairs = _sc_pairs(z32, iip, ijp)
  coeffs = _sc_gather(w_rows, pairs.reshape(_EP // _GCH, _GCH))

  xs = drp[:, 0].reshape(_GE, 1, _EB)
  ys = drp[:, 1].reshape(_GE, 1, _EB)
  zs = drp[:, 2].reshape(_GE, 1, _EB)
  ii3 = iip.reshape(_GE, 1, _EB)
  ij3 = ijp.reshape(_GE, 1, _EB)
  medge = _tc_edge(xs, ys, zs, ii3, ij3, coeffs)

  ij2 = ijp.reshape(_EP // _CH2, _CH2)
  acc = _sc_scatter(medge, ij2)

  acc_t = jnp.transpose(acc, (0, 2, 1)).reshape(2, _NRC, _GA * _AB_SUB, 128)
  out = _tc_contract(acc_t)
  return out.reshape(_NOUT, _AP).T[:_N_ATOMS]
